# baseline (speedup 1.0000x reference)
import functools

import jax
import jax.numpy as jnp
from jax import lax
from jax.experimental import pallas as pl
from jax.experimental.pallas import tpu as pltpu
from jax.experimental.pallas import tpu_sc as plsc

NUM_REL = 20000
NUM_CLASSES = 51
PAYLOAD_W = 128
ROW_BLOCK = 2000

N_PAD = 20480
SORT_W = 16
CHUNK = N_PAD // SORT_W
VREGS = CHUNK // 16
GATHER_W = 32
GCHUNK = N_PAD // GATHER_W
STAGE_ROWS = 1248
STAGE_TAIL = NUM_REL - SORT_W * STAGE_ROWS
N_PASS = 8

_SC_PARAMS = pltpu.CompilerParams(needs_layout_passes=False)


def _row_sum_xla_order(e):
    x = jnp.pad(e, ((0, 0), (0, (-e.shape[1]) % 8)))
    acc = x[:, 0:8]
    for k in range(1, x.shape[1] // 8):
        acc = acc + x[:, k * 8:(k + 1) * 8]
    g = 4
    while g >= 1:
        acc = acc[:, :g] + acc[:, g:2 * g]
        g //= 2
    return acc


def _softmax_body(logits_ref, pairs_ref, payload_ref, score_ref):
    x = logits_ref[...]
    m = jnp.max(x, axis=1, keepdims=True)
    e = jnp.exp(x - m)
    s = _row_sum_xla_order(e)
    p = e / s
    fg = p[:, 1:]
    score_ref[0, 0, :] = jnp.max(fg, axis=1)
    cls = jnp.argmax(fg, axis=1).astype(jnp.int32) + 1
    pair_bits = lax.bitcast_convert_type(pairs_ref[...], jnp.float32)
    cls_bits = lax.bitcast_convert_type(cls[:, None], jnp.float32)
    zeros = jnp.zeros((x.shape[0], PAYLOAD_W - NUM_CLASSES - 3), jnp.float32)
    payload_ref[...] = jnp.concatenate(
        [p, pair_bits, cls_bits, zeros], axis=1)


def _softmax_stage(rel_logits, rel_pairs):
    n_blocks = NUM_REL // ROW_BLOCK
    payload, score = pl.pallas_call(
        _softmax_body,
        grid=(n_blocks,),
        in_specs=[
            pl.BlockSpec((ROW_BLOCK, NUM_CLASSES), lambda i: (i, 0)),
            pl.BlockSpec((ROW_BLOCK, 2), lambda i: (i, 0)),
        ],
        out_specs=[
            pl.BlockSpec((ROW_BLOCK, PAYLOAD_W), lambda i: (i, 0)),
            pl.BlockSpec((1, 1, ROW_BLOCK), lambda i: (i, 0, 0)),
        ],
        out_shape=[
            jax.ShapeDtypeStruct((NUM_REL, PAYLOAD_W), jnp.float32),
            jax.ShapeDtypeStruct((n_blocks, 1, ROW_BLOCK), jnp.float32),
        ],
    )(rel_logits, rel_pairs)
    return payload, score.reshape(-1)


_GATHER_DN = lax.GatherDimensionNumbers(
    offset_dims=(), collapsed_slice_dims=(0,), start_index_map=(0,))


def _take16(x, idx):
    return lax.gather(x, idx[:, None], _GATHER_DN, slice_sizes=(1,),
                      mode=lax.GatherScatterMode.PROMISE_IN_BOUNDS)


def _sort_body(keys_hbm, vals_hbm, perm_hbm,
               kst, av, bv, hist,
               kb, vb, vs, ps, sd, sl, rk, cnt16, hv, ha, sem):
    w = lax.axis_index("s")
    cbase = w * CHUNK
    iota = lax.iota(jnp.int32, 16)

    pltpu.sync_copy(keys_hbm.at[pl.ds(cbase, CHUNK)],
                    kst.at[pl.ds(cbase, CHUNK)])
    plsc.subcore_barrier()

    def run_pass(p, src_v, dst_v):
        shift = 4 * p
        if p == 0:
            pltpu.sync_copy(vals_hbm.at[pl.ds(cbase, CHUNK)], vb)
        else:
            pltpu.sync_copy(src_v.at[pl.ds(cbase, CHUNK)], vb)
        pltpu.async_copy(kst.at[vb], kb, sem).wait()

        def hist_step(v, running):
            kv = kb[pl.ds(v * 16, 16)]
            dig = lax.shift_right_logical(kv, shift) & 15
            key16 = dig * 16 + iota
            srt = jnp.sort(key16)
            s_dig = lax.shift_right_logical(srt, 4)
            s_lane = srt & 15
            prev = _take16(s_dig, jnp.maximum(iota - 1, 0))
            runfirst = (iota == 0) | (s_dig != prev)
            start = plsc.cummax(jnp.where(runfirst, iota, 0))
            rank = iota - start
            nxt = _take16(s_dig, jnp.minimum(iota + 1, 15))
            runlast = (iota == 15) | (nxt != s_dig)
            cnt16[...] = jnp.zeros((16,), jnp.int32)
            plsc.store_scatter(cnt16, [s_dig], rank + 1, mask=runlast)
            sd[pl.ds(v * 16, 16)] = s_dig
            sl[pl.ds(v * 16, 16)] = s_lane
            rk[pl.ds(v * 16, 16)] = rank
            ps[pl.ds(v * 16, 16)] = running
            return running + cnt16[...]

        running = lax.fori_loop(0, VREGS, hist_step,
                                jnp.zeros((16,), jnp.int32))

        hv[...] = running
        pltpu.sync_copy(hv, hist.at[w])
        plsc.subcore_barrier()

        pltpu.sync_copy(hist, ha)
        tot = ha[0]
        before = jnp.where(0 < w, ha[0], 0)
        for r in range(1, SORT_W):
            row = ha[r]
            tot = tot + row
            before = before + jnp.where(r < w, row, 0)
        excl = plsc.cumsum(tot) - tot
        base = excl + before

        def perm_step(v, carry):
            vv = vb[pl.ds(v * 16, 16)]
            s_dig = sd[pl.ds(v * 16, 16)]
            s_lane = sl[pl.ds(v * 16, 16)]
            rank = rk[pl.ds(v * 16, 16)]
            pre = ps[pl.ds(v * 16, 16)]
            combined = base + pre
            pos = _take16(combined, s_dig) + rank
            vs[pl.ds(v * 16, 16)] = _take16(vv, s_lane)
            ps[pl.ds(v * 16, 16)] = pos
            return carry

        lax.fori_loop(0, VREGS, perm_step, jnp.int32(0))

        pltpu.async_copy(vs, dst_v.at[ps], sem).wait()
        plsc.subcore_barrier()

    for p in range(N_PASS):
        if p % 2 == 0:
            run_pass(p, av, bv)
        else:
            run_pass(p, bv, av)

    pltpu.sync_copy(av.at[pl.ds(cbase, CHUNK)], vb)
    pltpu.sync_copy(vb, perm_hbm.at[pl.ds(cbase, CHUNK)])


def _sort_stage(keys, vals):
    mesh = plsc.VectorSubcoreMesh(
        core_axis_name="c", subcore_axis_name="s", num_cores=1)
    f = functools.partial(
        pl.kernel,
        out_type=jax.ShapeDtypeStruct((N_PAD,), jnp.int32),
        mesh=mesh,
        scratch_types=[
            pltpu.VMEM_SHARED((N_PAD,), jnp.int32),
            pltpu.VMEM_SHARED((N_PAD,), jnp.int32),
            pltpu.VMEM_SHARED((N_PAD,), jnp.int32),
            pltpu.VMEM_SHARED((SORT_W, 16), jnp.int32),
            pltpu.VMEM((CHUNK,), jnp.int32),
            pltpu.VMEM((CHUNK,), jnp.int32),
            pltpu.VMEM((CHUNK,), jnp.int32),
            pltpu.VMEM((CHUNK,), jnp.int32),
            pltpu.VMEM((CHUNK,), jnp.int32),
            pltpu.VMEM((CHUNK,), jnp.int32),
            pltpu.VMEM((CHUNK,), jnp.int32),
            pltpu.VMEM((16,), jnp.int32),
            pltpu.VMEM((16,), jnp.int32),
            pltpu.VMEM((SORT_W, 16), jnp.int32),
            pltpu.SemaphoreType.DMA,
        ],
        compiler_params=_SC_PARAMS,
    )(_sort_body)
    return f(keys, vals)


def _gather_body(perm_hbm, payload_hbm, out_hbm,
                 idxa, idxb, rows, sem):
    s = lax.axis_index("s")
    c = lax.axis_index("c")
    wid = s * 2 + c
    base = wid * GCHUNK
    half = GCHUNK // 2
    pltpu.sync_copy(perm_hbm.at[pl.ds(base, half)], idxa)
    pltpu.sync_copy(perm_hbm.at[pl.ds(base + half, half)], idxb)
    pltpu.async_copy(payload_hbm.at[idxa], rows, sem).wait()
    pltpu.sync_copy(rows, out_hbm.at[pl.ds(base, half)])
    pltpu.async_copy(payload_hbm.at[idxb], rows, sem).wait()
    pltpu.sync_copy(rows, out_hbm.at[pl.ds(base + half, half)])


def _gather_stage(perm, payload):
    mesh = plsc.VectorSubcoreMesh(
        core_axis_name="c", subcore_axis_name="s", num_cores=2)
    f = functools.partial(
        pl.kernel,
        out_type=jax.ShapeDtypeStruct((N_PAD, PAYLOAD_W), jnp.float32),
        mesh=mesh,
        scratch_types=[
            pltpu.VMEM((GCHUNK // 2,), jnp.int32),
            pltpu.VMEM((GCHUNK // 2,), jnp.int32),
            pltpu.VMEM((GCHUNK // 2, PAYLOAD_W), jnp.float32),
            pltpu.SemaphoreType.DMA,
        ],
        compiler_params=_SC_PARAMS,
    )(_gather_body)
    return f(perm, payload)


def kernel(rel_logits, rel_pair_idxs):
    payload, score = _softmax_stage(rel_logits,
                                    rel_pair_idxs.astype(jnp.int32))

    bits = lax.bitcast_convert_type(score, jnp.uint32)
    keys = lax.bitcast_convert_type(jnp.bitwise_not(bits), jnp.int32)
    keys = jnp.concatenate(
        [keys, jnp.full((N_PAD - NUM_REL,), -1, jnp.int32)])
    vals = jnp.arange(N_PAD, dtype=jnp.int32)

    perm = _sort_stage(keys, vals)
    perm = jnp.minimum(perm, NUM_REL - 1)
    out = _gather_stage(perm, payload)

    outp = out[:NUM_REL, :NUM_CLASSES]
    outpair = lax.bitcast_convert_type(
        out[:NUM_REL, NUM_CLASSES:NUM_CLASSES + 2], jnp.int32)
    outc = lax.bitcast_convert_type(
        out[:NUM_REL, NUM_CLASSES + 2], jnp.int32)
    return (outpair, outp, outc)


# speedup vs baseline: 1.0242x; 1.0242x over previous
import functools

import jax
import jax.numpy as jnp
from jax import lax
from jax.experimental import pallas as pl
from jax.experimental.pallas import tpu as pltpu
from jax.experimental.pallas import tpu_sc as plsc

NUM_REL = 20000
NUM_CLASSES = 51
PAYLOAD_W = 128
ROW_BLOCK = 2000

N_PAD = 20480
SORT_W = 16
CHUNK = N_PAD // SORT_W
VREGS = CHUNK // 16
GATHER_W = 32
GCHUNK = N_PAD // GATHER_W
STAGE_ROWS = 1248
STAGE_TAIL = NUM_REL - SORT_W * STAGE_ROWS
N_PASS = 8

_SC_PARAMS = pltpu.CompilerParams(needs_layout_passes=False)


def _row_sum_xla_order(e):
    x = jnp.pad(e, ((0, 0), (0, (-e.shape[1]) % 8)))
    acc = x[:, 0:8]
    for k in range(1, x.shape[1] // 8):
        acc = acc + x[:, k * 8:(k + 1) * 8]
    g = 4
    while g >= 1:
        acc = acc[:, :g] + acc[:, g:2 * g]
        g //= 2
    return acc


def _softmax_body(logits_ref, pairs_ref, payload_ref, score_ref):
    x = logits_ref[...]
    m = jnp.max(x, axis=1, keepdims=True)
    e = jnp.exp(x - m)
    s = _row_sum_xla_order(e)
    p = e / s
    fg = p[:, 1:]
    score_ref[0, 0, :] = jnp.max(fg, axis=1)
    cls = jnp.argmax(fg, axis=1).astype(jnp.int32) + 1
    pair_bits = lax.bitcast_convert_type(pairs_ref[...], jnp.float32)
    cls_bits = lax.bitcast_convert_type(cls[:, None], jnp.float32)
    zeros = jnp.zeros((x.shape[0], PAYLOAD_W - NUM_CLASSES - 3), jnp.float32)
    payload_ref[...] = jnp.concatenate(
        [p, pair_bits, cls_bits, zeros], axis=1)


def _softmax_stage(rel_logits, rel_pairs):
    n_blocks = NUM_REL // ROW_BLOCK
    payload, score = pl.pallas_call(
        _softmax_body,
        grid=(n_blocks,),
        in_specs=[
            pl.BlockSpec((ROW_BLOCK, NUM_CLASSES), lambda i: (i, 0)),
            pl.BlockSpec((ROW_BLOCK, 2), lambda i: (i, 0)),
        ],
        out_specs=[
            pl.BlockSpec((ROW_BLOCK, PAYLOAD_W), lambda i: (i, 0)),
            pl.BlockSpec((1, 1, ROW_BLOCK), lambda i: (i, 0, 0)),
        ],
        out_shape=[
            jax.ShapeDtypeStruct((NUM_REL, PAYLOAD_W), jnp.float32),
            jax.ShapeDtypeStruct((n_blocks, 1, ROW_BLOCK), jnp.float32),
        ],
    )(rel_logits, rel_pairs)
    return payload, score.reshape(-1)


_GATHER_DN = lax.GatherDimensionNumbers(
    offset_dims=(), collapsed_slice_dims=(0,), start_index_map=(0,))


def _take16(x, idx):
    return lax.gather(x, idx[:, None], _GATHER_DN, slice_sizes=(1,),
                      mode=lax.GatherScatterMode.PROMISE_IN_BOUNDS)


def _sortgather_body(keys_hbm, vals_hbm, payload_hbm, out_hbm,
                     kst, av, bv, hist,
                     kb, vb, vs, ps, sd, sl, rk, cnt16, hv, ha,
                     idxa, idxb, rows, sem):
    w = lax.axis_index("s")
    cbase = w * CHUNK
    iota = lax.iota(jnp.int32, 16)

    pltpu.sync_copy(keys_hbm.at[pl.ds(cbase, CHUNK)],
                    kst.at[pl.ds(cbase, CHUNK)])
    plsc.subcore_barrier()

    def run_pass(p, src_v, dst_v):
        shift = 4 * p
        if p == 0:
            pltpu.sync_copy(vals_hbm.at[pl.ds(cbase, CHUNK)], vb)
        else:
            pltpu.sync_copy(src_v.at[pl.ds(cbase, CHUNK)], vb)
        pltpu.async_copy(kst.at[vb], kb, sem).wait()

        def hist_step(v, running):
            kv = kb[pl.ds(v * 16, 16)]
            dig = lax.shift_right_logical(kv, shift) & 15
            key16 = dig * 16 + iota
            srt = jnp.sort(key16)
            s_dig = lax.shift_right_logical(srt, 4)
            s_lane = srt & 15
            prev = _take16(s_dig, jnp.maximum(iota - 1, 0))
            runfirst = (iota == 0) | (s_dig != prev)
            start = plsc.cummax(jnp.where(runfirst, iota, 0))
            rank = iota - start
            nxt = _take16(s_dig, jnp.minimum(iota + 1, 15))
            runlast = (iota == 15) | (nxt != s_dig)
            cnt16[...] = jnp.zeros((16,), jnp.int32)
            plsc.store_scatter(cnt16, [s_dig], rank + 1, mask=runlast)
            sd[pl.ds(v * 16, 16)] = s_dig
            sl[pl.ds(v * 16, 16)] = s_lane
            rk[pl.ds(v * 16, 16)] = rank
            ps[pl.ds(v * 16, 16)] = running
            return running + cnt16[...]

        running = lax.fori_loop(0, VREGS, hist_step,
                                jnp.zeros((16,), jnp.int32))

        hv[...] = running
        pltpu.sync_copy(hv, hist.at[w])
        plsc.subcore_barrier()

        pltpu.sync_copy(hist, ha)
        tot = ha[0]
        before = jnp.where(0 < w, ha[0], 0)
        for r in range(1, SORT_W):
            row = ha[r]
            tot = tot + row
            before = before + jnp.where(r < w, row, 0)
        excl = plsc.cumsum(tot) - tot
        base = excl + before

        def perm_step(v, carry):
            vv = vb[pl.ds(v * 16, 16)]
            s_dig = sd[pl.ds(v * 16, 16)]
            s_lane = sl[pl.ds(v * 16, 16)]
            rank = rk[pl.ds(v * 16, 16)]
            pre = ps[pl.ds(v * 16, 16)]
            combined = base + pre
            pos = _take16(combined, s_dig) + rank
            vs[pl.ds(v * 16, 16)] = _take16(vv, s_lane)
            ps[pl.ds(v * 16, 16)] = pos
            return carry

        lax.fori_loop(0, VREGS, perm_step, jnp.int32(0))

        pltpu.async_copy(vs, dst_v.at[ps], sem).wait()
        plsc.subcore_barrier()

    for p in range(N_PASS):
        if p % 2 == 0:
            run_pass(p, av, bv)
        else:
            run_pass(p, bv, av)

    c = lax.axis_index("c")
    wid = c * SORT_W + w
    base = wid * GCHUNK
    half = GCHUNK // 2
    pltpu.sync_copy(av.at[pl.ds(base, half)], idxa)
    pltpu.sync_copy(av.at[pl.ds(base + half, half)], idxb)

    def clamp_step(v, carry):
        idxa[pl.ds(v * 16, 16)] = jnp.minimum(
            idxa[pl.ds(v * 16, 16)], NUM_REL - 1)
        idxb[pl.ds(v * 16, 16)] = jnp.minimum(
            idxb[pl.ds(v * 16, 16)], NUM_REL - 1)
        return carry

    lax.fori_loop(0, half // 16, clamp_step, jnp.int32(0))

    pltpu.async_copy(payload_hbm.at[idxa], rows, sem).wait()
    pltpu.sync_copy(rows, out_hbm.at[pl.ds(base, half)])
    pltpu.async_copy(payload_hbm.at[idxb], rows, sem).wait()
    pltpu.sync_copy(rows, out_hbm.at[pl.ds(base + half, half)])


def _sortgather_stage(keys, vals, payload):
    mesh = plsc.VectorSubcoreMesh(
        core_axis_name="c", subcore_axis_name="s", num_cores=2)
    f = functools.partial(
        pl.kernel,
        out_type=jax.ShapeDtypeStruct((N_PAD, PAYLOAD_W), jnp.float32),
        mesh=mesh,
        scratch_types=[
            pltpu.VMEM_SHARED((N_PAD,), jnp.int32),
            pltpu.VMEM_SHARED((N_PAD,), jnp.int32),
            pltpu.VMEM_SHARED((N_PAD,), jnp.int32),
            pltpu.VMEM_SHARED((SORT_W, 16), jnp.int32),
            pltpu.VMEM((CHUNK,), jnp.int32),
            pltpu.VMEM((CHUNK,), jnp.int32),
            pltpu.VMEM((CHUNK,), jnp.int32),
            pltpu.VMEM((CHUNK,), jnp.int32),
            pltpu.VMEM((CHUNK,), jnp.int32),
            pltpu.VMEM((CHUNK,), jnp.int32),
            pltpu.VMEM((CHUNK,), jnp.int32),
            pltpu.VMEM((16,), jnp.int32),
            pltpu.VMEM((16,), jnp.int32),
            pltpu.VMEM((SORT_W, 16), jnp.int32),
            pltpu.VMEM((GCHUNK // 2,), jnp.int32),
            pltpu.VMEM((GCHUNK // 2,), jnp.int32),
            pltpu.VMEM((GCHUNK // 2, PAYLOAD_W), jnp.float32),
            pltpu.SemaphoreType.DMA,
        ],
        compiler_params=_SC_PARAMS,
    )(_sortgather_body)
    return f(keys, vals, payload)


def kernel(rel_logits, rel_pair_idxs):
    payload, score = _softmax_stage(rel_logits,
                                    rel_pair_idxs.astype(jnp.int32))

    bits = lax.bitcast_convert_type(score, jnp.uint32)
    keys = lax.bitcast_convert_type(jnp.bitwise_not(bits), jnp.int32)
    keys = jnp.concatenate(
        [keys, jnp.full((N_PAD - NUM_REL,), -1, jnp.int32)])
    vals = jnp.arange(N_PAD, dtype=jnp.int32)

    out = _sortgather_stage(keys, vals, payload)

    outp = out[:NUM_REL, :NUM_CLASSES]
    outpair = lax.bitcast_convert_type(
        out[:NUM_REL, NUM_CLASSES:NUM_CLASSES + 2], jnp.int32)
    outc = lax.bitcast_convert_type(
        out[:NUM_REL, NUM_CLASSES + 2], jnp.int32)
    return (outpair, outp, outc)
